# baseline (speedup 1.0000x reference)
import functools

import jax
import jax.numpy as jnp
from jax import lax
from jax.experimental import pallas as pl
from jax.experimental.pallas import tpu as pltpu
from jax.experimental.pallas import tpu_sc as plsc

_H = 256
_V = 100000
_B = 2048
_K = 128
_NC = 2
_NS = 16
_NW = _NC * _NS
_APW = _B // _NW
_VBLK = 4000
_L = 16


def _tc_transform_body(emb_ref, w2_ref, b2_ref, stims_ref, w1_ref, b1_ref,
                       t_ref, x_ref):
    t_ref[...] = jnp.maximum(
        jnp.dot(emb_ref[...], w2_ref[...],
                preferred_element_type=jnp.float32) + b2_ref[...], 0.0)

    @pl.when(pl.program_id(0) == 0)
    def _():
        x_ref[...] = jnp.maximum(
            jnp.dot(stims_ref[...], w1_ref[...],
                    preferred_element_type=jnp.float32) + b1_ref[...], 0.0)


def _tc_transform(embed_table, W2, b2, stims, W1, b1, *, interpret=False):
    return pl.pallas_call(
        _tc_transform_body,
        grid=(_V // _VBLK,),
        in_specs=[
            pl.BlockSpec((_VBLK, _H), lambda i: (i, 0)),
            pl.BlockSpec((_H, _H), lambda i: (0, 0)),
            pl.BlockSpec((1, _H), lambda i: (0, 0)),
            pl.BlockSpec((_B, _H), lambda i: (0, 0)),
            pl.BlockSpec((_H, _H), lambda i: (0, 0)),
            pl.BlockSpec((1, _H), lambda i: (0, 0)),
        ],
        out_specs=[
            pl.BlockSpec((_VBLK, _H), lambda i: (i, 0)),
            pl.BlockSpec((_B, _H), lambda i: (0, 0)),
        ],
        out_shape=[
            jax.ShapeDtypeStruct((_V, _H), jnp.float32),
            jax.ShapeDtypeStruct((_B, _H), jnp.float32),
        ],
        interpret=interpret,
    )(embed_table, W2, b2, stims, W1, b1)


_KH = _K // 2


def _sc_body(t_hbm, x_hbm, nm_hbm, atn_hbm, scores_hbm, idx_hbm,
             atn_v, names_v, x_v, rows_v, sc_v, ix_v,
             sem_n, sem_r0, sem_r1, sem_r2, sem_r3):
    wid = lax.axis_index("s") * _NC + lax.axis_index("c")
    base = wid * _APW
    pltpu.sync_copy(atn_hbm.at[pl.ds(base, _APW)], atn_v)
    pltpu.sync_copy(x_hbm.at[pl.ds(base, _APW)], x_v)
    lanes = lax.iota(jnp.int32, _L)
    sems = [sem_r0, sem_r1, sem_r2, sem_r3]

    def name_chunk(ch, carry):
        for u in range(8):
            a = ch * 8 + u
            pltpu.async_copy(nm_hbm.at[atn_v.at[a]], names_v.at[a], sem_n)
        for u in range(8):
            a = ch * 8 + u
            pltpu.make_async_copy(nm_hbm.at[atn_v.at[a]], names_v.at[a],
                                  sem_n).wait()
        return carry

    lax.fori_loop(0, _APW // 8, name_chunk, 0)

    def compute_half(a, half, slot, kcarry):
        xc = [x_v[a, pl.ds(c * _L, _L)] for c in range(_H // _L)]

        def kg_body(kg, kc):
            bval, bidx = kc
            kbase = half * _KH + kg * _L
            sv = jnp.zeros((_L,), jnp.float32)
            for kk in range(_L):
                k = kg * _L + kk
                acc = xc[0] * rows_v[slot, k, pl.ds(0, _L)]
                for c in range(1, _H // _L):
                    acc = acc + xc[c] * rows_v[slot, k, pl.ds(c * _L, _L)]
                s = jnp.sum(acc)
                sv = jnp.where(lanes == kk, s, sv)
            sc_v[a, pl.ds(kbase, _L)] = sv
            upd = sv > bval
            bval = jnp.where(upd, sv, bval)
            bidx = jnp.where(upd, kbase + lanes, bidx)
            return bval, bidx

        return lax.fori_loop(0, _KH // _L, kg_body, kcarry)

    def finalize(a, kcarry):
        bval, bidx = kcarry
        m = jnp.max(bval)
        cand = jnp.where(bval == m, bidx, jnp.int32(_K))
        amax = jnp.min(cand)
        ix_v[a, ...] = jnp.full((_L,), amax, jnp.int32)

    def half_gather(a, half, slot):
        idx = names_v.at[a, pl.ds(half * _KH, _KH)]
        pltpu.async_copy(t_hbm.at[idx], rows_v.at[slot], sems[slot])

    def half_wait(a, half, slot):
        idx = names_v.at[a, pl.ds(half * _KH, _KH)]
        pltpu.make_async_copy(t_hbm.at[idx], rows_v.at[slot],
                              sems[slot]).wait()

    def init_carry():
        return (jnp.full((_L,), -1.0, jnp.float32),
                jnp.zeros((_L,), jnp.int32))

    half_gather(0, 0, 0)
    half_gather(0, 1, 1)
    half_gather(1, 0, 2)
    half_gather(1, 1, 3)

    def pair_body(p, carry):
        a0 = 2 * p
        a1 = a0 + 1
        more = p + 1 < _APW // 2

        half_wait(a0, 0, 0)
        c0 = compute_half(a0, 0, 0, init_carry())

        @pl.when(more)
        def _():
            half_gather(a0 + 2, 0, 0)

        half_wait(a0, 1, 1)
        c0 = compute_half(a0, 1, 1, c0)
        finalize(a0, c0)

        @pl.when(more)
        def _():
            half_gather(a0 + 2, 1, 1)

        half_wait(a1, 0, 2)
        c1 = compute_half(a1, 0, 2, init_carry())

        @pl.when(more)
        def _():
            half_gather(a1 + 2, 0, 2)

        half_wait(a1, 1, 3)
        c1 = compute_half(a1, 1, 3, c1)
        finalize(a1, c1)

        @pl.when(more)
        def _():
            half_gather(a1 + 2, 1, 3)

        return carry

    lax.fori_loop(0, _APW // 2, pair_body, 0)
    pltpu.sync_copy(sc_v, scores_hbm.at[pl.ds(base, _APW)])
    pltpu.sync_copy(ix_v, idx_hbm.at[pl.ds(base, _APW)])


def _sc_classify(T, X, name_map, atn, *, interpret=False):
    mesh = plsc.VectorSubcoreMesh(core_axis_name="c", subcore_axis_name="s",
                                  num_cores=_NC, num_subcores=_NS)
    fn = pl.kernel(
        _sc_body,
        mesh=mesh,
        out_type=(
            jax.ShapeDtypeStruct((_B, _K), jnp.float32),
            jax.ShapeDtypeStruct((_B, _L), jnp.int32),
        ),
        scratch_types=[
            pltpu.VMEM((_APW, _K), jnp.int32),
            pltpu.VMEM((_APW, _K), jnp.int32),
            pltpu.VMEM((_APW, _H), jnp.float32),
            pltpu.VMEM((4, _KH, _H), jnp.float32),
            pltpu.VMEM((_APW, _K), jnp.float32),
            pltpu.VMEM((_APW, _L), jnp.int32),
            pltpu.SemaphoreType.DMA,
            pltpu.SemaphoreType.DMA,
            pltpu.SemaphoreType.DMA,
            pltpu.SemaphoreType.DMA,
            pltpu.SemaphoreType.DMA,
        ],
        compiler_params=pltpu.CompilerParams(needs_layout_passes=False),
        interpret=interpret,
    )
    return fn(T, X, name_map, atn)


def kernel(stims, embed_table, name_map, atn_tensor, W1, b1, W2, b2):
    i, j, k, _n = atn_tensor.shape
    T, X = _tc_transform(embed_table, W2, b2.reshape(1, _H), stims, W1,
                         b1.reshape(1, _H))
    atn = atn_tensor.reshape(_B, _K)
    scores, idx16 = _sc_classify(T, X, name_map, atn)
    return scores.reshape(i, j, k), idx16[:, :1].reshape(i, j)


# speedup vs baseline: 1.3216x; 1.3216x over previous
import functools

import jax
import jax.numpy as jnp
from jax import lax
from jax.experimental import pallas as pl
from jax.experimental.pallas import tpu as pltpu
from jax.experimental.pallas import tpu_sc as plsc

_H = 256
_V = 100000
_B = 2048
_K = 128
_NC = 2
_NS = 16
_NW = _NC * _NS
_APW = _B // _NW
_VBLK = 4000
_L = 16


def _tc_transform_body(emb_ref, w2_ref, b2_ref, stims_ref, w1_ref, b1_ref,
                       t_ref, x_ref):
    t_ref[...] = jnp.maximum(
        jnp.dot(emb_ref[...], w2_ref[...],
                preferred_element_type=jnp.float32) + b2_ref[...], 0.0)

    @pl.when(pl.program_id(0) == 0)
    def _():
        x_ref[...] = jnp.maximum(
            jnp.dot(stims_ref[...], w1_ref[...],
                    preferred_element_type=jnp.float32) + b1_ref[...], 0.0)


def _tc_transform(embed_table, W2, b2, stims, W1, b1, *, interpret=False):
    return pl.pallas_call(
        _tc_transform_body,
        grid=(_V // _VBLK,),
        in_specs=[
            pl.BlockSpec((_VBLK, _H), lambda i: (i, 0)),
            pl.BlockSpec((_H, _H), lambda i: (0, 0)),
            pl.BlockSpec((1, _H), lambda i: (0, 0)),
            pl.BlockSpec((_B, _H), lambda i: (0, 0)),
            pl.BlockSpec((_H, _H), lambda i: (0, 0)),
            pl.BlockSpec((1, _H), lambda i: (0, 0)),
        ],
        out_specs=[
            pl.BlockSpec((_VBLK, _H), lambda i: (i, 0)),
            pl.BlockSpec((_B, _H), lambda i: (0, 0)),
        ],
        out_shape=[
            jax.ShapeDtypeStruct((_V, _H), jnp.float32),
            jax.ShapeDtypeStruct((_B, _H), jnp.float32),
        ],
        interpret=interpret,
    )(embed_table, W2, b2, stims, W1, b1)


_XCH = 8
_SCH = 16


def _sc_body(t_hbm, x_hbm, nm_hbm, atn_hbm, scores_hbm, idx_hbm,
             atn_c, names_v, xb, rows_v, sc_h, ix_v,
             sem_n, sem_x, sem_r0, sem_r1, sem_r2):
    wid = lax.axis_index("s") * _NC + lax.axis_index("c")
    base = wid * _APW
    lanes = lax.iota(jnp.int32, _L)
    sems = [sem_r0, sem_r1, sem_r2]

    def x_chunk_copy(g, slot):
        src = x_hbm.at[pl.ds(pl.multiple_of(base + g * _XCH, 8), _XCH)]
        return pltpu.make_async_copy(src, xb.at[slot], sem_x)

    x_chunk_copy(0, 0).start()

    def name_chunk(ch, carry):
        pltpu.sync_copy(atn_hbm.at[pl.ds(base + ch * 8, 8)], atn_c)
        for u in range(8):
            pltpu.async_copy(nm_hbm.at[atn_c.at[u]], names_v.at[ch * 8 + u],
                             sem_n)
        for u in range(8):
            pltpu.make_async_copy(nm_hbm.at[atn_c.at[u]],
                                  names_v.at[ch * 8 + u], sem_n).wait()
        return carry

    lax.fori_loop(0, _APW // 8, name_chunk, 0)

    def compute(a, slot):
        xs = lax.rem(lax.div(a, _XCH), 2)
        xr = lax.rem(a, _XCH)
        xc = [xb[xs, xr, pl.ds(c * _L, _L)] for c in range(_H // _L)]

        def kg_body(kg, kc):
            bval, bidx = kc
            sv = jnp.zeros((_L,), jnp.float32)
            for kk in range(_L):
                k = kg * _L + kk
                acc = xc[0] * rows_v[slot, k, pl.ds(0, _L)]
                for c in range(1, _H // _L):
                    acc = acc + xc[c] * rows_v[slot, k, pl.ds(c * _L, _L)]
                s = jnp.sum(acc)
                sv = jnp.where(lanes == kk, s, sv)
            sc_h[lax.rem(a, _SCH), pl.ds(kg * _L, _L)] = sv
            upd = sv > bval
            bval = jnp.where(upd, sv, bval)
            bidx = jnp.where(upd, kg * _L + lanes, bidx)
            return bval, bidx

        bval0 = jnp.full((_L,), -1.0, jnp.float32)
        bidx0 = jnp.zeros((_L,), jnp.int32)
        bval, bidx = lax.fori_loop(0, _K // _L, kg_body, (bval0, bidx0))
        m = jnp.max(bval)
        cand = jnp.where(bval == m, bidx, jnp.int32(_K))
        amax = jnp.min(cand)
        ix_v[a, ...] = jnp.full((_L,), amax, jnp.int32)
        @pl.when(lax.rem(a, _SCH) == _SCH - 1)
        def _():
            blk_base = pl.multiple_of(base + (a - (_SCH - 1)), 8)
            pltpu.sync_copy(sc_h, scores_hbm.at[pl.ds(blk_base, _SCH)])

    def row_gather(a, slot):
        pltpu.async_copy(t_hbm.at[names_v.at[a]], rows_v.at[slot], sems[slot])

    def row_wait(a, slot):
        pltpu.make_async_copy(t_hbm.at[names_v.at[a]], rows_v.at[slot],
                              sems[slot]).wait()

    row_gather(0, 0)
    row_gather(1, 1)
    row_gather(2, 2)

    def agent_body(a, carry):
        u = lax.rem(a, 3)

        @pl.when(lax.rem(a, _XCH) == 0)
        def _():
            g = lax.div(a, _XCH)
            x_chunk_copy(g, lax.rem(g, 2)).wait()

            @pl.when(a + _XCH < _APW)
            def _():
                x_chunk_copy(g + 1, lax.rem(g + 1, 2)).start()

        for s in range(3):
            @pl.when(u == s)
            def _():
                row_wait(a, s)

        compute(a, u)

        @pl.when(a + 3 < _APW)
        def _():
            for s in range(3):
                @pl.when(u == s)
                def _():
                    row_gather(a + 3, s)

        return carry

    lax.fori_loop(0, _APW, agent_body, 0)
    pltpu.sync_copy(ix_v, idx_hbm.at[pl.ds(base, _APW)])


def _sc_classify(T, X, name_map, atn, *, interpret=False):
    mesh = plsc.VectorSubcoreMesh(core_axis_name="c", subcore_axis_name="s",
                                  num_cores=_NC, num_subcores=_NS)
    fn = pl.kernel(
        _sc_body,
        mesh=mesh,
        out_type=(
            jax.ShapeDtypeStruct((_B, _K), jnp.float32),
            jax.ShapeDtypeStruct((_B, _L), jnp.int32),
        ),
        scratch_types=[
            pltpu.VMEM((8, _K), jnp.int32),
            pltpu.VMEM((_APW, _K), jnp.int32),
            pltpu.VMEM((2, _XCH, _H), jnp.float32),
            pltpu.VMEM((3, _K, _H), jnp.float32),
            pltpu.VMEM((_SCH, _K), jnp.float32),
            pltpu.VMEM((_APW, _L), jnp.int32),
            pltpu.SemaphoreType.DMA,
            pltpu.SemaphoreType.DMA,
            pltpu.SemaphoreType.DMA,
            pltpu.SemaphoreType.DMA,
            pltpu.SemaphoreType.DMA,
        ],
        compiler_params=pltpu.CompilerParams(needs_layout_passes=False),
        interpret=interpret,
    )
    return fn(T, X, name_map, atn)


def kernel(stims, embed_table, name_map, atn_tensor, W1, b1, W2, b2):
    i, j, k, _n = atn_tensor.shape
    T, X = _tc_transform(embed_table, W2, b2.reshape(1, _H), stims, W1,
                         b1.reshape(1, _H))
    atn = atn_tensor.reshape(_B, _K)
    scores, idx16 = _sc_classify(T, X, name_map, atn)
    return scores.reshape(i, j, k), idx16[:, :1].reshape(i, j)


# speedup vs baseline: 1.3487x; 1.0205x over previous
import functools

import jax
import jax.numpy as jnp
from jax import lax
from jax.experimental import pallas as pl
from jax.experimental.pallas import tpu as pltpu
from jax.experimental.pallas import tpu_sc as plsc

_H = 256
_V = 100000
_B = 2048
_K = 128
_NC = 2
_NS = 16
_NW = _NC * _NS
_APW = _B // _NW
_VBLK = 4000
_L = 16


def _tc_transform_body(emb_ref, w2_ref, b2_ref, stims_ref, w1_ref, b1_ref,
                       t_ref, x_ref):
    t_ref[...] = jnp.maximum(
        jnp.dot(emb_ref[...], w2_ref[...],
                preferred_element_type=jnp.float32) + b2_ref[...], 0.0)

    @pl.when(pl.program_id(0) == 0)
    def _():
        x_ref[...] = jnp.maximum(
            jnp.dot(stims_ref[...], w1_ref[...],
                    preferred_element_type=jnp.float32) + b1_ref[...], 0.0)


def _tc_transform(embed_table, W2, b2, stims, W1, b1, *, interpret=False):
    return pl.pallas_call(
        _tc_transform_body,
        grid=(_V // _VBLK,),
        in_specs=[
            pl.BlockSpec((_VBLK, _H), lambda i: (i, 0)),
            pl.BlockSpec((_H, _H), lambda i: (0, 0)),
            pl.BlockSpec((1, _H), lambda i: (0, 0)),
            pl.BlockSpec((_B, _H), lambda i: (0, 0)),
            pl.BlockSpec((_H, _H), lambda i: (0, 0)),
            pl.BlockSpec((1, _H), lambda i: (0, 0)),
        ],
        out_specs=[
            pl.BlockSpec((_VBLK, _H), lambda i: (i, 0)),
            pl.BlockSpec((_B, _H), lambda i: (0, 0)),
        ],
        out_shape=[
            jax.ShapeDtypeStruct((_V, _H), jnp.float32),
            jax.ShapeDtypeStruct((_B, _H), jnp.float32),
        ],
        interpret=interpret,
    )(embed_table, W2, b2, stims, W1, b1)


_XCH = 8
_SCH = 16


def _sc_names_body(nm_hbm, atn_hbm, names_out, atn_v, names_v, sem_n):
    wid = lax.axis_index("s") * _NC + lax.axis_index("c")
    base = wid * _APW
    pltpu.sync_copy(atn_hbm.at[pl.ds(base, _APW)], atn_v)

    def name_chunk(ch, carry):
        for u in range(8):
            pltpu.async_copy(nm_hbm.at[atn_v.at[ch * 8 + u]],
                             names_v.at[ch * 8 + u], sem_n)
        for u in range(8):
            pltpu.make_async_copy(nm_hbm.at[atn_v.at[ch * 8 + u]],
                                  names_v.at[ch * 8 + u], sem_n).wait()
        return carry

    lax.fori_loop(0, _APW // 8, name_chunk, 0)
    pltpu.sync_copy(names_v, names_out.at[pl.ds(base, _APW)])


def _sc_names(name_map, atn):
    mesh = plsc.VectorSubcoreMesh(core_axis_name="c", subcore_axis_name="s",
                                  num_cores=_NC, num_subcores=_NS)
    fn = pl.kernel(
        _sc_names_body,
        mesh=mesh,
        out_type=jax.ShapeDtypeStruct((_B, _K), jnp.int32),
        scratch_types=[
            pltpu.VMEM((_APW, _K), jnp.int32),
            pltpu.VMEM((_APW, _K), jnp.int32),
            pltpu.SemaphoreType.DMA,
        ],
        compiler_params=pltpu.CompilerParams(needs_layout_passes=False),
    )
    return fn(name_map, atn)


def _sc_body(t_hbm, x_hbm, names_hbm, scores_hbm, idx_hbm,
             names_v, xb, rows_v, sc_h, ix_v,
             sem_n, sem_x, sem_r0, sem_r1, sem_r2):
    wid = lax.axis_index("s") * _NC + lax.axis_index("c")
    base = wid * _APW
    lanes = lax.iota(jnp.int32, _L)
    sems = [sem_r0, sem_r1, sem_r2]

    def x_chunk_copy(g, slot):
        src = x_hbm.at[pl.ds(pl.multiple_of(base + g * _XCH, 8), _XCH)]
        return pltpu.make_async_copy(src, xb.at[slot], sem_x)

    x_chunk_copy(0, 0).start()
    pltpu.async_copy(names_hbm.at[pl.ds(base, _APW)], names_v, sem_n).wait()

    def compute(a, slot):
        xs = lax.rem(lax.div(a, _XCH), 2)
        xr = lax.rem(a, _XCH)
        xc = [xb[xs, xr, pl.ds(c * _L, _L)] for c in range(_H // _L)]

        def kg_body(kg, kc):
            bval, bidx = kc
            sv = jnp.zeros((_L,), jnp.float32)
            for kk in range(_L):
                k = kg * _L + kk
                acc = xc[0] * rows_v[slot, k, pl.ds(0, _L)]
                for c in range(1, _H // _L):
                    acc = acc + xc[c] * rows_v[slot, k, pl.ds(c * _L, _L)]
                s = jnp.sum(acc)
                sv = jnp.where(lanes == kk, s, sv)
            sc_h[lax.rem(a, _SCH), pl.ds(kg * _L, _L)] = sv
            upd = sv > bval
            bval = jnp.where(upd, sv, bval)
            bidx = jnp.where(upd, kg * _L + lanes, bidx)
            return bval, bidx

        bval0 = jnp.full((_L,), -1.0, jnp.float32)
        bidx0 = jnp.zeros((_L,), jnp.int32)
        bval, bidx = lax.fori_loop(0, _K // _L, kg_body, (bval0, bidx0))
        m = jnp.max(bval)
        cand = jnp.where(bval == m, bidx, jnp.int32(_K))
        amax = jnp.min(cand)
        ix_v[a, ...] = jnp.full((_L,), amax, jnp.int32)
        @pl.when(lax.rem(a, _SCH) == _SCH - 1)
        def _():
            blk_base = pl.multiple_of(base + (a - (_SCH - 1)), 8)
            pltpu.sync_copy(sc_h, scores_hbm.at[pl.ds(blk_base, _SCH)])

    def row_gather(a, slot):
        pltpu.async_copy(t_hbm.at[names_v.at[a]], rows_v.at[slot], sems[slot])

    def row_wait(a, slot):
        pltpu.make_async_copy(t_hbm.at[names_v.at[a]], rows_v.at[slot],
                              sems[slot]).wait()

    row_gather(0, 0)
    row_gather(1, 1)
    row_gather(2, 2)

    def agent_body(a, carry):
        u = lax.rem(a, 3)

        @pl.when(lax.rem(a, _XCH) == 0)
        def _():
            g = lax.div(a, _XCH)
            x_chunk_copy(g, lax.rem(g, 2)).wait()

            @pl.when(a + _XCH < _APW)
            def _():
                x_chunk_copy(g + 1, lax.rem(g + 1, 2)).start()

        for s in range(3):
            @pl.when(u == s)
            def _():
                row_wait(a, s)

        compute(a, u)

        @pl.when(a + 3 < _APW)
        def _():
            for s in range(3):
                @pl.when(u == s)
                def _():
                    row_gather(a + 3, s)

        return carry

    lax.fori_loop(0, _APW, agent_body, 0)
    pltpu.sync_copy(ix_v, idx_hbm.at[pl.ds(base, _APW)])


def _sc_classify(T, X, names, *, interpret=False):
    mesh = plsc.VectorSubcoreMesh(core_axis_name="c", subcore_axis_name="s",
                                  num_cores=_NC, num_subcores=_NS)
    fn = pl.kernel(
        _sc_body,
        mesh=mesh,
        out_type=(
            jax.ShapeDtypeStruct((_B, _K), jnp.float32),
            jax.ShapeDtypeStruct((_B, _L), jnp.int32),
        ),
        scratch_types=[
            pltpu.VMEM((_APW, _K), jnp.int32),
            pltpu.VMEM((2, _XCH, _H), jnp.float32),
            pltpu.VMEM((3, _K, _H), jnp.float32),
            pltpu.VMEM((_SCH, _K), jnp.float32),
            pltpu.VMEM((_APW, _L), jnp.int32),
            pltpu.SemaphoreType.DMA,
            pltpu.SemaphoreType.DMA,
            pltpu.SemaphoreType.DMA,
            pltpu.SemaphoreType.DMA,
            pltpu.SemaphoreType.DMA,
        ],
        compiler_params=pltpu.CompilerParams(needs_layout_passes=False),
        interpret=interpret,
    )
    return fn(T, X, names)


def kernel(stims, embed_table, name_map, atn_tensor, W1, b1, W2, b2):
    i, j, k, _n = atn_tensor.shape
    T, X = _tc_transform(embed_table, W2, b2.reshape(1, _H), stims, W1,
                         b1.reshape(1, _H))
    atn = atn_tensor.reshape(_B, _K)
    names = _sc_names(name_map, atn)
    scores, idx16 = _sc_classify(T, X, names)
    return scores.reshape(i, j, k), idx16[:, :1].reshape(i, j)


# speedup vs baseline: 1.3501x; 1.0010x over previous
import jax
import jax.numpy as jnp
from jax import lax
from jax.experimental import pallas as pl
from jax.experimental.pallas import tpu as pltpu
from jax.experimental.pallas import tpu_sc as plsc

_H = 256
_V = 100000
_B = 2048
_K = 128
_NC = 2
_NS = 16
_NW = _NC * _NS
_APW = _B // _NW
_VBLK = 4000
_L = 16


def _tc_transform_body(emb_ref, w2_ref, b2_ref, stims_ref, w1_ref, b1_ref,
                       t_ref, x_ref):
    t_ref[...] = jnp.maximum(
        jnp.dot(emb_ref[...], w2_ref[...],
                preferred_element_type=jnp.float32) + b2_ref[...], 0.0)

    @pl.when(pl.program_id(0) == 0)
    def _():
        x_ref[...] = jnp.maximum(
            jnp.dot(stims_ref[...], w1_ref[...],
                    preferred_element_type=jnp.float32) + b1_ref[...], 0.0)


def _tc_transform(embed_table, W2, b2, stims, W1, b1):
    return pl.pallas_call(
        _tc_transform_body,
        grid=(_V // _VBLK,),
        in_specs=[
            pl.BlockSpec((_VBLK, _H), lambda i: (i, 0)),
            pl.BlockSpec((_H, _H), lambda i: (0, 0)),
            pl.BlockSpec((1, _H), lambda i: (0, 0)),
            pl.BlockSpec((_B, _H), lambda i: (0, 0)),
            pl.BlockSpec((_H, _H), lambda i: (0, 0)),
            pl.BlockSpec((1, _H), lambda i: (0, 0)),
        ],
        out_specs=[
            pl.BlockSpec((_VBLK, _H), lambda i: (i, 0)),
            pl.BlockSpec((_B, _H), lambda i: (0, 0)),
        ],
        out_shape=[
            jax.ShapeDtypeStruct((_V, _H), jnp.float32),
            jax.ShapeDtypeStruct((_B, _H), jnp.float32),
        ],
    )(embed_table, W2, b2, stims, W1, b1)


_XCH = 8
_SCH = 16


def _sc_names_body(nm_hbm, atn_hbm, names_out, atn_v, names_v, sem_n):
    wid = lax.axis_index("s") * _NC + lax.axis_index("c")
    base = wid * _APW
    pltpu.sync_copy(atn_hbm.at[pl.ds(base, _APW)], atn_v)

    def name_chunk(ch, carry):
        for u in range(8):
            pltpu.async_copy(nm_hbm.at[atn_v.at[ch * 8 + u]],
                             names_v.at[ch * 8 + u], sem_n)
        for u in range(8):
            pltpu.make_async_copy(nm_hbm.at[atn_v.at[ch * 8 + u]],
                                  names_v.at[ch * 8 + u], sem_n).wait()
        return carry

    lax.fori_loop(0, _APW // 8, name_chunk, 0)
    pltpu.sync_copy(names_v, names_out.at[pl.ds(base, _APW)])


def _sc_names(name_map, atn):
    mesh = plsc.VectorSubcoreMesh(core_axis_name="c", subcore_axis_name="s",
                                  num_cores=_NC, num_subcores=_NS)
    fn = pl.kernel(
        _sc_names_body,
        mesh=mesh,
        out_type=jax.ShapeDtypeStruct((_B, _K), jnp.int32),
        scratch_types=[
            pltpu.VMEM((_APW, _K), jnp.int32),
            pltpu.VMEM((_APW, _K), jnp.int32),
            pltpu.SemaphoreType.DMA,
        ],
        compiler_params=pltpu.CompilerParams(needs_layout_passes=False),
    )
    return fn(name_map, atn)


def _sc_body(t_hbm, x_hbm, names_hbm, scores_hbm, idx_hbm,
             names_v, xb, rows_v, sc_h, ix_v,
             sem_n, sem_x, sem_r0, sem_r1, sem_r2):
    wid = lax.axis_index("s") * _NC + lax.axis_index("c")
    base = wid * _APW
    lanes = lax.iota(jnp.int32, _L)
    sems = [sem_r0, sem_r1, sem_r2]

    def x_chunk_copy(g, slot):
        src = x_hbm.at[pl.ds(pl.multiple_of(base + g * _XCH, 8), _XCH)]
        return pltpu.make_async_copy(src, xb.at[slot], sem_x)

    x_chunk_copy(0, 0).start()
    pltpu.async_copy(names_hbm.at[pl.ds(base, _APW)], names_v, sem_n).wait()

    def compute(a, slot):
        xs = lax.rem(lax.div(a, _XCH), 2)
        xr = lax.rem(a, _XCH)
        xc = [xb[xs, xr, pl.ds(c * _L, _L)] for c in range(_H // _L)]

        def kg_body(kg, kc):
            bval, bidx = kc
            sv = jnp.zeros((_L,), jnp.float32)
            for kk in range(_L):
                k = kg * _L + kk
                acc = xc[0] * rows_v[slot, k, pl.ds(0, _L)]
                for c in range(1, _H // _L):
                    acc = acc + xc[c] * rows_v[slot, k, pl.ds(c * _L, _L)]
                s = jnp.sum(acc)
                sv = jnp.where(lanes == kk, s, sv)
            sc_h[lax.rem(a, _SCH), pl.ds(kg * _L, _L)] = sv
            upd = sv > bval
            bval = jnp.where(upd, sv, bval)
            bidx = jnp.where(upd, kg * _L + lanes, bidx)
            return bval, bidx

        bval0 = jnp.full((_L,), -1.0, jnp.float32)
        bidx0 = jnp.zeros((_L,), jnp.int32)
        bval, bidx = lax.fori_loop(0, _K // _L, kg_body, (bval0, bidx0))
        m = jnp.max(bval)
        cand = jnp.where(bval == m, bidx, jnp.int32(_K))
        amax = jnp.min(cand)
        ix_v[a, ...] = jnp.full((_L,), amax, jnp.int32)
        @pl.when(lax.rem(a, _SCH) == _SCH - 1)
        def _():
            blk_base = pl.multiple_of(base + (a - (_SCH - 1)), 8)
            pltpu.sync_copy(sc_h, scores_hbm.at[pl.ds(blk_base, _SCH)])

    def row_gather(a, slot):
        pltpu.async_copy(t_hbm.at[names_v.at[a]], rows_v.at[slot], sems[slot])

    def row_wait(a, slot):
        pltpu.make_async_copy(t_hbm.at[names_v.at[a]], rows_v.at[slot],
                              sems[slot]).wait()

    row_gather(0, 0)
    row_gather(1, 1)
    row_gather(2, 2)

    def agent_body(a, carry):
        u = lax.rem(a, 3)

        @pl.when(lax.rem(a, _XCH) == 0)
        def _():
            g = lax.div(a, _XCH)
            x_chunk_copy(g, lax.rem(g, 2)).wait()

            @pl.when(a + _XCH < _APW)
            def _():
                x_chunk_copy(g + 1, lax.rem(g + 1, 2)).start()

        for s in range(3):
            @pl.when(u == s)
            def _():
                row_wait(a, s)

        compute(a, u)

        @pl.when(a + 3 < _APW)
        def _():
            for s in range(3):
                @pl.when(u == s)
                def _():
                    row_gather(a + 3, s)

        return carry

    lax.fori_loop(0, _APW, agent_body, 0)
    pltpu.sync_copy(ix_v, idx_hbm.at[pl.ds(base, _APW)])


def _sc_classify(T, X, names):
    mesh = plsc.VectorSubcoreMesh(core_axis_name="c", subcore_axis_name="s",
                                  num_cores=_NC, num_subcores=_NS)
    fn = pl.kernel(
        _sc_body,
        mesh=mesh,
        out_type=(
            jax.ShapeDtypeStruct((_B, _K), jnp.float32),
            jax.ShapeDtypeStruct((_B, _L), jnp.int32),
        ),
        scratch_types=[
            pltpu.VMEM((_APW, _K), jnp.int32),
            pltpu.VMEM((2, _XCH, _H), jnp.float32),
            pltpu.VMEM((3, _K, _H), jnp.float32),
            pltpu.VMEM((_SCH, _K), jnp.float32),
            pltpu.VMEM((_APW, _L), jnp.int32),
            pltpu.SemaphoreType.DMA,
            pltpu.SemaphoreType.DMA,
            pltpu.SemaphoreType.DMA,
            pltpu.SemaphoreType.DMA,
            pltpu.SemaphoreType.DMA,
        ],
        compiler_params=pltpu.CompilerParams(needs_layout_passes=False),
    )
    return fn(T, X, names)


def kernel(stims, embed_table, name_map, atn_tensor, W1, b1, W2, b2):
    i, j, k, _n = atn_tensor.shape
    T, X = _tc_transform(embed_table, W2, b2.reshape(1, _H), stims, W1,
                         b1.reshape(1, _H))
    atn = atn_tensor.reshape(_B, _K)
    names = _sc_names(name_map, atn)
    scores, idx16 = _sc_classify(T, X, names)
    return scores.reshape(i, j, k), idx16[:, :1].reshape(i, j)
